# baseline (speedup 1.0000x reference)
import functools

import jax
import jax.numpy as jnp
from jax import lax
from jax.experimental import pallas as pl
from jax.experimental.pallas import tpu as pltpu
from jax.experimental.pallas import tpu_sc as plsc

T = 2048
D = 768
E = 8
FF = 3072
BM = 256
NB = (2 * T) // BM + E
P = NB * BM
NW = 32
LANES = 16


def _router_body(x_ref, wg_ref, pos1_ref, pos2_ref, w1b_ref, w2b_ref,
                 bexp_ref, bval_ref, aux_ref):
    x = x_ref[...]
    logits = lax.dot_general(x, wg_ref[...], (((1,), (1,)), ((), ())),
                             preferred_element_type=jnp.float32)
    m = jnp.max(logits, axis=1, keepdims=True)
    ex = jnp.exp(logits - m)
    scores = ex / jnp.sum(ex, axis=1, keepdims=True)
    lane = lax.broadcasted_iota(jnp.int32, (T, E), 1)
    m1 = jnp.max(scores, axis=1, keepdims=True)
    a1 = jnp.min(jnp.where(scores == m1, lane, E), axis=1, keepdims=True)
    s2 = jnp.where(lane == a1, -jnp.inf, scores)
    m2 = jnp.max(s2, axis=1, keepdims=True)
    a2 = jnp.min(jnp.where(s2 == m2, lane, E), axis=1, keepdims=True)
    denom = m1 + m2
    w1b_ref[...] = jnp.broadcast_to(m1 / denom, (T, LANES))
    w2b_ref[...] = jnp.broadcast_to(m2 / denom, (T, LANES))
    oh1 = (lane == a1).astype(jnp.float32)
    oh2 = (lane == a2).astype(jnp.float32)

    c1 = jnp.sum(oh1, axis=0)
    c2 = jnp.sum(oh2, axis=0)
    p1 = jnp.sum(oh1 * scores, axis=0)
    p2 = jnp.sum(oh2 * scores, axis=0)
    aux_ref[0, 0] = (jnp.sum(p1 * c1) + jnp.sum(p2 * c2)) * (float(E) / float(T))

    oh = oh1 + oh2
    cex = jnp.concatenate([jnp.zeros((1, E), jnp.float32), oh[:-1]], axis=0)
    sh = 1
    while sh < T:
        cex = cex + jnp.concatenate(
            [jnp.zeros((sh, E), jnp.float32), cex[:-sh]], axis=0)
        sh *= 2
    counts = jnp.sum(oh, axis=0, keepdims=True)
    padc = jnp.ceil(counts / BM) * BM
    er = lax.broadcasted_iota(jnp.int32, (E, E), 0)
    ec = lax.broadcasted_iota(jnp.int32, (E, E), 1)
    tril = (er < ec).astype(jnp.float32)
    start = lax.dot_general(padc, tril, (((1,), (0,)), ((), ())),
                            preferred_element_type=jnp.float32)
    slot = cex + start
    pos1_ref[...] = jnp.sum(slot * oh1, axis=1, keepdims=True).astype(jnp.int32)
    pos2_ref[...] = jnp.sum(slot * oh2, axis=1, keepdims=True).astype(jnp.int32)

    brow = lax.broadcasted_iota(jnp.int32, (NB, E), 0).astype(jnp.float32) * BM
    scol = jnp.broadcast_to(start, (NB, E))
    bexp_ref[...] = (jnp.sum((scol <= brow).astype(jnp.int32), axis=1,
                             keepdims=True) - 1)
    bval_ref[...] = (brow[:, :1] < jnp.broadcast_to(jnp.sum(padc), (NB, 1))
                     ).astype(jnp.int32)


def _router(flat, Wg):
    return pl.pallas_call(
        _router_body,
        grid=(1,),
        in_specs=[
            pl.BlockSpec((T, D), lambda i: (0, 0)),
            pl.BlockSpec((E, D), lambda i: (0, 0)),
        ],
        out_specs=[
            pl.BlockSpec((T, 1), lambda i: (0, 0)),
            pl.BlockSpec((T, 1), lambda i: (0, 0)),
            pl.BlockSpec((T, LANES), lambda i: (0, 0)),
            pl.BlockSpec((T, LANES), lambda i: (0, 0)),
            pl.BlockSpec((NB, 1), lambda i: (0, 0)),
            pl.BlockSpec((NB, 1), lambda i: (0, 0)),
            pl.BlockSpec((1, 1), lambda i: (0, 0), memory_space=pltpu.SMEM),
        ],
        out_shape=[
            jax.ShapeDtypeStruct((T, 1), jnp.int32),
            jax.ShapeDtypeStruct((T, 1), jnp.int32),
            jax.ShapeDtypeStruct((T, LANES), jnp.float32),
            jax.ShapeDtypeStruct((T, LANES), jnp.float32),
            jax.ShapeDtypeStruct((NB, 1), jnp.int32),
            jax.ShapeDtypeStruct((NB, 1), jnp.int32),
            jax.ShapeDtypeStruct((1, 1), jnp.float32),
        ],
    )(flat, Wg)


@functools.lru_cache(maxsize=None)
def _sc_mesh():
    return plsc.VectorSubcoreMesh(core_axis_name="c", subcore_axis_name="s")

_TOK_PER_W = T // NW


def _dispatch_sc(x_hbm, pos1_hbm, pos2_hbm, gx_hbm,
                 idx1_v, idx2_v, rows_v, sem):
    wid = lax.axis_index("s") * 2 + lax.axis_index("c")
    base = wid * _TOK_PER_W
    pltpu.sync_copy(pos1_hbm.at[pl.ds(base, _TOK_PER_W)], idx1_v)
    pltpu.sync_copy(pos2_hbm.at[pl.ds(base, _TOK_PER_W)], idx2_v)
    pltpu.sync_copy(x_hbm.at[pl.ds(base, _TOK_PER_W)], rows_v)
    cp1 = pltpu.async_copy(rows_v, gx_hbm.at[idx1_v], sem)
    cp2 = pltpu.async_copy(rows_v, gx_hbm.at[idx2_v], sem)
    cp1.wait()
    cp2.wait()


def _dispatch(flat, pos1, pos2):
    f = pl.kernel(
        _dispatch_sc,
        out_type=jax.ShapeDtypeStruct((P, D), jnp.float32),
        mesh=_sc_mesh(),
        scratch_types=[
            pltpu.VMEM((_TOK_PER_W,), jnp.int32),
            pltpu.VMEM((_TOK_PER_W,), jnp.int32),
            pltpu.VMEM((_TOK_PER_W, D), jnp.float32),
            pltpu.SemaphoreType.DMA,
        ],
        compiler_params=pltpu.CompilerParams(needs_layout_passes=False),
    )
    return f(flat, pos1, pos2)


FH = FF // 2


def _gmm_body(bexp_ref, bval_ref, gx_ref, w1_ref, w2_ref, out_ref):
    i = pl.program_id(0)
    j = pl.program_id(1)

    @pl.when(bval_ref[i] != 0)
    def _compute():
        h = lax.dot_general(gx_ref[...], w1_ref[0], (((1,), (1,)), ((), ())),
                            preferred_element_type=jnp.float32,
                            precision=lax.Precision.DEFAULT)
        h = h * lax.logistic(h)
        part = lax.dot_general(h, w2_ref[0], (((1,), (1,)), ((), ())),
                               preferred_element_type=jnp.float32,
                               precision=lax.Precision.DEFAULT)

        @pl.when(j == 0)
        def _init():
            out_ref[...] = part

        @pl.when(j != 0)
        def _acc():
            out_ref[...] = out_ref[...] + part


def _group_mlp(gx, bexp, bval, W1, W2):
    grid_spec = pltpu.PrefetchScalarGridSpec(
        num_scalar_prefetch=2,
        grid=(NB, 2),
        in_specs=[
            pl.BlockSpec((BM, D), lambda i, j, be, bv: (i, 0)),
            pl.BlockSpec((1, FH, D), lambda i, j, be, bv: (be[i], j, 0)),
            pl.BlockSpec((1, D, FH), lambda i, j, be, bv: (be[i], 0, j)),
        ],
        out_specs=pl.BlockSpec((BM, D), lambda i, j, be, bv: (i, 0)),
    )
    return pl.pallas_call(
        _gmm_body,
        grid_spec=grid_spec,
        out_shape=jax.ShapeDtypeStruct((P, D), jnp.float32),
    )(bexp, bval, gx, W1, W2)


def _combine_sc(gout_hbm, pos1_hbm, pos2_hbm, w1b_hbm, w2b_hbm, y_hbm,
                idx1_v, idx2_v, w1_v, w2_v, rows1_v, rows2_v, sem):
    wid = lax.axis_index("s") * 2 + lax.axis_index("c")
    base = wid * _TOK_PER_W
    pltpu.sync_copy(pos1_hbm.at[pl.ds(base, _TOK_PER_W)], idx1_v)
    pltpu.sync_copy(pos2_hbm.at[pl.ds(base, _TOK_PER_W)], idx2_v)
    pltpu.sync_copy(w1b_hbm.at[pl.ds(base, _TOK_PER_W)], w1_v)
    pltpu.sync_copy(w2b_hbm.at[pl.ds(base, _TOK_PER_W)], w2_v)
    cp1 = pltpu.async_copy(gout_hbm.at[idx1_v], rows1_v, sem)
    cp2 = pltpu.async_copy(gout_hbm.at[idx2_v], rows2_v, sem)
    cp1.wait()
    cp2.wait()

    def _wadd(r, _):
        wa = w1_v[r, :]
        wb = w2_v[r, :]
        for c in range(D // LANES):
            sl = pl.ds(c * LANES, LANES)
            rows1_v[r, sl] = rows1_v[r, sl] * wa + rows2_v[r, sl] * wb
        return 0
    lax.fori_loop(0, _TOK_PER_W, _wadd, 0)
    pltpu.sync_copy(rows1_v, y_hbm.at[pl.ds(base, _TOK_PER_W)])


def _combine(gout, pos1, pos2, w1b, w2b):
    f = pl.kernel(
        _combine_sc,
        out_type=jax.ShapeDtypeStruct((T, D), jnp.float32),
        mesh=_sc_mesh(),
        scratch_types=[
            pltpu.VMEM((_TOK_PER_W,), jnp.int32),
            pltpu.VMEM((_TOK_PER_W,), jnp.int32),
            pltpu.VMEM((_TOK_PER_W, LANES), jnp.float32),
            pltpu.VMEM((_TOK_PER_W, LANES), jnp.float32),
            pltpu.VMEM((_TOK_PER_W, D), jnp.float32),
            pltpu.VMEM((_TOK_PER_W, D), jnp.float32),
            pltpu.SemaphoreType.DMA,
        ],
        compiler_params=pltpu.CompilerParams(needs_layout_passes=False),
    )
    return f(gout, pos1, pos2, w1b, w2b)


@functools.partial(jax.jit, static_argnames=())
def kernel(x, Wg, W1, W2):
    b, s, d = x.shape
    flat = x.reshape(T, D)
    pos1, pos2, w1b, w2b, bexp, bval, aux = _router(flat, Wg)
    pos1 = pos1.reshape(T)
    pos2 = pos2.reshape(T)
    gx = _dispatch(flat, pos1, pos2)
    gout = _group_mlp(gx, bexp.reshape(NB), bval.reshape(NB), W1, W2)
    y = _combine(gout, pos1, pos2, w1b, w2b)
    return y.reshape(b, s, d), aux.reshape(())


# speedup vs baseline: 1.3109x; 1.3109x over previous
import functools

import jax
import jax.numpy as jnp
from jax import lax
from jax.experimental import pallas as pl
from jax.experimental.pallas import tpu as pltpu
from jax.experimental.pallas import tpu_sc as plsc

T = 2048
D = 768
E = 8
FF = 3072
BM = 256
NB = (2 * T) // BM + E
P = NB * BM
NW = 32
LANES = 16


def _router_body(x_ref, wg_ref, pos1_ref, pos2_ref, w1b_ref, w2b_ref,
                 bexp_ref, bval_ref, aux_ref):
    x = x_ref[...]
    logits = lax.dot_general(x, wg_ref[...], (((1,), (1,)), ((), ())),
                             preferred_element_type=jnp.float32)
    m = jnp.max(logits, axis=1, keepdims=True)
    ex = jnp.exp(logits - m)
    scores = ex / jnp.sum(ex, axis=1, keepdims=True)
    lane = lax.broadcasted_iota(jnp.int32, (T, E), 1)
    m1 = jnp.max(scores, axis=1, keepdims=True)
    a1 = jnp.min(jnp.where(scores == m1, lane, E), axis=1, keepdims=True)
    s2 = jnp.where(lane == a1, -jnp.inf, scores)
    m2 = jnp.max(s2, axis=1, keepdims=True)
    a2 = jnp.min(jnp.where(s2 == m2, lane, E), axis=1, keepdims=True)
    denom = m1 + m2
    w1b_ref[...] = jnp.broadcast_to(m1 / denom, (T, LANES))
    w2b_ref[...] = jnp.broadcast_to(m2 / denom, (T, LANES))
    oh1 = (lane == a1).astype(jnp.float32)
    oh2 = (lane == a2).astype(jnp.float32)

    c1 = jnp.sum(oh1, axis=0)
    c2 = jnp.sum(oh2, axis=0)
    p1 = jnp.sum(oh1 * scores, axis=0)
    p2 = jnp.sum(oh2 * scores, axis=0)
    aux_ref[0, 0] = (jnp.sum(p1 * c1) + jnp.sum(p2 * c2)) * (float(E) / float(T))

    oh = oh1 + oh2
    cex = jnp.concatenate([jnp.zeros((1, E), jnp.float32), oh[:-1]], axis=0)
    sh = 1
    while sh < T:
        cex = cex + jnp.concatenate(
            [jnp.zeros((sh, E), jnp.float32), cex[:-sh]], axis=0)
        sh *= 2
    counts = jnp.sum(oh, axis=0, keepdims=True)
    padc = jnp.ceil(counts / BM) * BM
    er = lax.broadcasted_iota(jnp.int32, (E, E), 0)
    ec = lax.broadcasted_iota(jnp.int32, (E, E), 1)
    tril = (er < ec).astype(jnp.float32)
    start = lax.dot_general(padc, tril, (((1,), (0,)), ((), ())),
                            preferred_element_type=jnp.float32)
    slot = cex + start
    pos1_ref[...] = jnp.sum(slot * oh1, axis=1, keepdims=True).astype(jnp.int32)
    pos2_ref[...] = jnp.sum(slot * oh2, axis=1, keepdims=True).astype(jnp.int32)

    brow = lax.broadcasted_iota(jnp.int32, (NB, E), 0).astype(jnp.float32) * BM
    scol = jnp.broadcast_to(start, (NB, E))
    bexp_ref[...] = (jnp.sum((scol <= brow).astype(jnp.int32), axis=1,
                             keepdims=True) - 1)
    bval_ref[...] = (brow[:, :1] < jnp.broadcast_to(jnp.sum(padc), (NB, 1))
                     ).astype(jnp.int32)


def _router(flat, Wg):
    return pl.pallas_call(
        _router_body,
        grid=(1,),
        in_specs=[
            pl.BlockSpec((T, D), lambda i: (0, 0)),
            pl.BlockSpec((E, D), lambda i: (0, 0)),
        ],
        out_specs=[
            pl.BlockSpec((T, 1), lambda i: (0, 0)),
            pl.BlockSpec((T, 1), lambda i: (0, 0)),
            pl.BlockSpec((T, LANES), lambda i: (0, 0)),
            pl.BlockSpec((T, LANES), lambda i: (0, 0)),
            pl.BlockSpec((NB, 1), lambda i: (0, 0)),
            pl.BlockSpec((NB, 1), lambda i: (0, 0)),
            pl.BlockSpec((1, 1), lambda i: (0, 0), memory_space=pltpu.SMEM),
        ],
        out_shape=[
            jax.ShapeDtypeStruct((T, 1), jnp.int32),
            jax.ShapeDtypeStruct((T, 1), jnp.int32),
            jax.ShapeDtypeStruct((T, LANES), jnp.float32),
            jax.ShapeDtypeStruct((T, LANES), jnp.float32),
            jax.ShapeDtypeStruct((NB, 1), jnp.int32),
            jax.ShapeDtypeStruct((NB, 1), jnp.int32),
            jax.ShapeDtypeStruct((1, 1), jnp.float32),
        ],
    )(flat, Wg)


@functools.lru_cache(maxsize=None)
def _sc_mesh():
    return plsc.VectorSubcoreMesh(core_axis_name="c", subcore_axis_name="s")

_TOK_PER_W = T // NW


def _dispatch_sc(x_hbm, pos1_hbm, pos2_hbm, gx_hbm,
                 idx1_v, idx2_v, rows_v, sem):
    wid = lax.axis_index("s") * 2 + lax.axis_index("c")
    base = wid * _TOK_PER_W
    pltpu.sync_copy(pos1_hbm.at[pl.ds(base, _TOK_PER_W)], idx1_v)
    pltpu.sync_copy(pos2_hbm.at[pl.ds(base, _TOK_PER_W)], idx2_v)
    pltpu.sync_copy(x_hbm.at[pl.ds(base, _TOK_PER_W)], rows_v)
    cp1 = pltpu.async_copy(rows_v, gx_hbm.at[idx1_v], sem)
    cp2 = pltpu.async_copy(rows_v, gx_hbm.at[idx2_v], sem)
    cp1.wait()
    cp2.wait()


def _dispatch(flat, pos1, pos2):
    f = pl.kernel(
        _dispatch_sc,
        out_type=jax.ShapeDtypeStruct((P, D), jnp.float32),
        mesh=_sc_mesh(),
        scratch_types=[
            pltpu.VMEM((_TOK_PER_W,), jnp.int32),
            pltpu.VMEM((_TOK_PER_W,), jnp.int32),
            pltpu.VMEM((_TOK_PER_W, D), jnp.float32),
            pltpu.SemaphoreType.DMA,
        ],
        compiler_params=pltpu.CompilerParams(needs_layout_passes=False),
    )
    return f(flat, pos1, pos2)


def _gmm_body(bexp_ref, bval_ref, gx_ref, w1_ref, w2_ref, out_ref):
    i = pl.program_id(0)

    @pl.when(bval_ref[i] != 0)
    def _compute():
        h = lax.dot_general(gx_ref[...], w1_ref[0], (((1,), (1,)), ((), ())),
                            preferred_element_type=jnp.float32,
                            precision=lax.Precision.DEFAULT)
        h = h * lax.logistic(h)
        out_ref[...] = lax.dot_general(h, w2_ref[0], (((1,), (1,)), ((), ())),
                                       preferred_element_type=jnp.float32,
                                       precision=lax.Precision.DEFAULT)


def _group_mlp(gx, bexp, bval, W1, W2):
    grid_spec = pltpu.PrefetchScalarGridSpec(
        num_scalar_prefetch=2,
        grid=(NB,),
        in_specs=[
            pl.BlockSpec((BM, D), lambda i, be, bv: (i, 0)),
            pl.BlockSpec((1, FF, D), lambda i, be, bv: (be[i], 0, 0)),
            pl.BlockSpec((1, D, FF), lambda i, be, bv: (be[i], 0, 0)),
        ],
        out_specs=pl.BlockSpec((BM, D), lambda i, be, bv: (i, 0)),
    )
    return pl.pallas_call(
        _gmm_body,
        grid_spec=grid_spec,
        out_shape=jax.ShapeDtypeStruct((P, D), jnp.float32),
    )(bexp, bval, gx, W1, W2)


def _combine_sc(gout_hbm, pos1_hbm, pos2_hbm, w1b_hbm, w2b_hbm, y_hbm,
                idx1_v, idx2_v, w1_v, w2_v, rows1_v, rows2_v, sem):
    wid = lax.axis_index("s") * 2 + lax.axis_index("c")
    base = wid * _TOK_PER_W
    pltpu.sync_copy(pos1_hbm.at[pl.ds(base, _TOK_PER_W)], idx1_v)
    pltpu.sync_copy(pos2_hbm.at[pl.ds(base, _TOK_PER_W)], idx2_v)
    pltpu.sync_copy(w1b_hbm.at[pl.ds(base, _TOK_PER_W)], w1_v)
    pltpu.sync_copy(w2b_hbm.at[pl.ds(base, _TOK_PER_W)], w2_v)
    cp1 = pltpu.async_copy(gout_hbm.at[idx1_v], rows1_v, sem)
    cp2 = pltpu.async_copy(gout_hbm.at[idx2_v], rows2_v, sem)
    cp1.wait()
    cp2.wait()

    def _wadd(r, _):
        wa = w1_v[r, :]
        wb = w2_v[r, :]
        for c in range(D // LANES):
            sl = pl.ds(c * LANES, LANES)
            rows1_v[r, sl] = rows1_v[r, sl] * wa + rows2_v[r, sl] * wb
        return 0
    lax.fori_loop(0, _TOK_PER_W, _wadd, 0)
    pltpu.sync_copy(rows1_v, y_hbm.at[pl.ds(base, _TOK_PER_W)])


def _combine(gout, pos1, pos2, w1b, w2b):
    f = pl.kernel(
        _combine_sc,
        out_type=jax.ShapeDtypeStruct((T, D), jnp.float32),
        mesh=_sc_mesh(),
        scratch_types=[
            pltpu.VMEM((_TOK_PER_W,), jnp.int32),
            pltpu.VMEM((_TOK_PER_W,), jnp.int32),
            pltpu.VMEM((_TOK_PER_W, LANES), jnp.float32),
            pltpu.VMEM((_TOK_PER_W, LANES), jnp.float32),
            pltpu.VMEM((_TOK_PER_W, D), jnp.float32),
            pltpu.VMEM((_TOK_PER_W, D), jnp.float32),
            pltpu.SemaphoreType.DMA,
        ],
        compiler_params=pltpu.CompilerParams(needs_layout_passes=False),
    )
    return f(gout, pos1, pos2, w1b, w2b)


@functools.partial(jax.jit, static_argnames=())
def kernel(x, Wg, W1, W2):
    b, s, d = x.shape
    flat = x.reshape(T, D)
    pos1, pos2, w1b, w2b, bexp, bval, aux = _router(flat, Wg)
    pos1 = pos1.reshape(T)
    pos2 = pos2.reshape(T)
    gx = _dispatch(flat, pos1, pos2)
    gout = _group_mlp(gx, bexp.reshape(NB), bval.reshape(NB), W1, W2)
    y = _combine(gout, pos1, pos2, w1b, w2b)
    return y.reshape(b, s, d), aux.reshape(())
